# Initial kernel scaffold; baseline (speedup 1.0000x reference)
import jax
import jax.numpy as jnp
from jax.experimental import pallas as pl


def kernel(h, e, p, edge_index, A1_W, A1_b, A2_W, A2_b, B1_W, B1_b, B2_W, B2_b, B3_W, B3_b, C1_W, C1_b, C2_W, C2_b, bn_h_g, bn_h_b, bn_e_g, bn_e_b):
    raise NotImplementedError("write your pallas kernel here")


# speedup vs baseline: 1.0093x; 1.0093x over previous
import functools

import jax
import jax.numpy as jnp
from jax.experimental import pallas as pl
from jax.experimental.pallas import tpu as pltpu

N, E, D = 10000, 320000, 128


def _node_pre_body(h_ref, p_ref, w_ref, b_ref, out_ref):
    h = h_ref[...]
    p = p_ref[...]
    def mm(x, i):
        return jax.lax.dot_general(
            x, w_ref[i], (((1,), (0,)), ((), ())),
            preferred_element_type=jnp.float32) + b_ref[i]
    a1 = mm(h, 0)
    b1 = mm(h, 1)
    b2 = mm(h, 2)
    c1 = mm(p, 3)
    v = mm(h, 4) + jax.lax.dot_general(
        p, w_ref[5], (((1,), (0,)), ((), ())),
        preferred_element_type=jnp.float32)
    cp = mm(p, 6)
    out_ref[0] = a1
    out_ref[1] = b1
    out_ref[2] = b2
    out_ref[3] = c1
    out_ref[4] = v
    out_ref[5] = cp


def _node_precompute(h, p, W, B):
    blk = 2000
    grid = (N // blk,)
    return pl.pallas_call(
        _node_pre_body,
        grid=grid,
        in_specs=[
            pl.BlockSpec((blk, D), lambda i: (i, 0)),
            pl.BlockSpec((blk, D), lambda i: (i, 0)),
            pl.BlockSpec((7, D, D), lambda i: (0, 0, 0)),
            pl.BlockSpec((7, 1, D), lambda i: (0, 0, 0)),
        ],
        out_specs=pl.BlockSpec((6, blk, D), lambda i: (0, i, 0)),
        out_shape=jax.ShapeDtypeStruct((6, N, D), jnp.float32),
    )(h, p, W, B)


def _edge_mm_body(e_ref, w_ref, b_ref, out_ref):
    out_ref[...] = jax.lax.dot_general(
        e_ref[...], w_ref[...], (((1,), (0,)), ((), ())),
        preferred_element_type=jnp.float32) + b_ref[...]


def _edge_matmul(e, W, b):
    blk = 4000
    return pl.pallas_call(
        _edge_mm_body,
        grid=(E // blk,),
        in_specs=[
            pl.BlockSpec((blk, D), lambda i: (i, 0)),
            pl.BlockSpec((D, D), lambda i: (0, 0)),
            pl.BlockSpec((1, D), lambda i: (0, 0)),
        ],
        out_specs=pl.BlockSpec((blk, D), lambda i: (i, 0)),
        out_shape=jax.ShapeDtypeStruct((E, D), jnp.float32),
    )(e, W, b.reshape(1, D))


def _estats_body(x_ref, out_ref):
    i = pl.program_id(0)

    @pl.when(i == 0)
    def _():
        out_ref[...] = jnp.zeros_like(out_ref)

    x = x_ref[...]
    s = jnp.sum(x, axis=0, keepdims=True)
    s2 = jnp.sum(x * x, axis=0, keepdims=True)
    out_ref[...] += jnp.concatenate([s, s2], axis=0)


def _edge_stats(hat):
    blk = 8000
    return pl.pallas_call(
        _estats_body,
        grid=(E // blk,),
        in_specs=[pl.BlockSpec((blk, D), lambda i: (i, 0))],
        out_specs=pl.BlockSpec((2, D), lambda i: (0, 0)),
        out_shape=jax.ShapeDtypeStruct((2, D), jnp.float32),
    )(hat)


def _eapply_body(x_ref, st_ref, g_ref, b_ref, out_ref):
    m = st_ref[0:1] * (1.0 / E)
    var = st_ref[1:2] * (1.0 / E) - m * m
    inv = jax.lax.rsqrt(var + 1e-5)
    out_ref[...] = jnp.maximum((x_ref[...] - m) * inv * g_ref[...] + b_ref[...], 0.0)


def _edge_apply(hat, stats, g, b):
    blk = 8000
    return pl.pallas_call(
        _eapply_body,
        grid=(E // blk,),
        in_specs=[
            pl.BlockSpec((blk, D), lambda i: (i, 0)),
            pl.BlockSpec((2, D), lambda i: (0, 0)),
            pl.BlockSpec((1, D), lambda i: (0, 0)),
            pl.BlockSpec((1, D), lambda i: (0, 0)),
        ],
        out_specs=pl.BlockSpec((blk, D), lambda i: (i, 0)),
        out_shape=jax.ShapeDtypeStruct((E, D), jnp.float32),
    )(hat, stats, g.reshape(1, D), b.reshape(1, D))


def _nodefin_body(a1_ref, c1_ref, s_ref, sv_ref, sp_ref, g_ref, b_ref, h_ref, p_ref):
    denom = s_ref[...] + 1e-6
    h_pre = a1_ref[...] + sv_ref[...] / denom
    p_pre = c1_ref[...] + sp_ref[...] / denom
    m = jnp.mean(h_pre, axis=0, keepdims=True)
    var = jnp.mean(h_pre * h_pre, axis=0, keepdims=True) - m * m
    inv = jax.lax.rsqrt(var + 1e-5)
    h_ref[...] = jnp.maximum((h_pre - m) * inv * g_ref[...] + b_ref[...], 0.0)
    p_ref[...] = jnp.tanh(p_pre)


def _node_finalize(A1_h, C1p, S, SV, SP, g, b):
    return pl.pallas_call(
        _nodefin_body,
        in_specs=[pl.BlockSpec((N, D), lambda: (0, 0))] * 5
        + [pl.BlockSpec((1, D), lambda: (0, 0))] * 2,
        out_specs=[pl.BlockSpec((N, D), lambda: (0, 0))] * 2,
        out_shape=[jax.ShapeDtypeStruct((N, D), jnp.float32)] * 2,
    )(A1_h, C1p, S, SV, SP, g.reshape(1, D), b.reshape(1, D))


@jax.jit
def kernel(h, e, p, edge_index, A1_W, A1_b, A2_W, A2_b, B1_W, B1_b, B2_W, B2_b,
           B3_W, B3_b, C1_W, C1_b, C2_W, C2_b, bn_h_g, bn_h_b, bn_e_g, bn_e_b):
    src = edge_index[0]
    dst = edge_index[1]

    W = jnp.stack([A1_W, B1_W, B2_W, C1_W, A2_W[:D], A2_W[D:], C2_W])
    B = jnp.stack([A1_b, B1_b, B2_b, C1_b, A2_b, jnp.zeros_like(A2_b), C2_b])[:, None, :]

    nodes = _node_precompute(h, p, W, B)
    A1_h, B1n, B2n, C1p, V, Cp = (nodes[i] for i in range(6))

    B3e = _edge_matmul(e, B3_W, B3_b)

    hat = B1n[src] + B2n[dst] + B3e
    sig = jax.nn.sigmoid(hat)
    S = jax.ops.segment_sum(sig, dst, num_segments=N)
    SV = jax.ops.segment_sum(sig * V[src], dst, num_segments=N)
    SP = jax.ops.segment_sum(sig * Cp[src], dst, num_segments=N)

    stats = _edge_stats(hat)
    e_new = _edge_apply(hat, stats, bn_e_g, bn_e_b)
    h_new, p_new = _node_finalize(A1_h, C1p, S, SV, SP, bn_h_g, bn_h_b)
    return (h_new, e_new, p_new)


# speedup vs baseline: 1.8064x; 1.7897x over previous
import functools

import jax
import jax.numpy as jnp
from jax.experimental import pallas as pl
from jax.experimental.pallas import tpu as pltpu
from jax.experimental.pallas import tpu_sc as plsc

N, E, D = 10000, 320000, 128
NC, NS = 2, 16
NW = NC * NS
NV = D // 16
NH = N // 2
PAD = 5120

_MESH = plsc.VectorSubcoreMesh(core_axis_name="c", subcore_axis_name="s")


def _node_pre_body(h_ref, p_ref, w_ref, b_ref, out_ref):
    h = h_ref[...]
    p = p_ref[...]

    def mm(x, i):
        return jax.lax.dot_general(
            x, w_ref[i], (((1,), (0,)), ((), ())),
            preferred_element_type=jnp.float32) + b_ref[i]

    out_ref[0] = mm(h, 0)
    out_ref[1] = mm(h, 1)
    out_ref[2] = mm(h, 2)
    out_ref[3] = mm(p, 3)
    out_ref[4] = mm(h, 4) + jax.lax.dot_general(
        p, w_ref[5], (((1,), (0,)), ((), ())),
        preferred_element_type=jnp.float32)
    out_ref[5] = mm(p, 6)


def _node_precompute(h, p, W, B):
    blk = 2000
    return pl.pallas_call(
        _node_pre_body,
        grid=(N // blk,),
        in_specs=[
            pl.BlockSpec((blk, D), lambda i: (i, 0)),
            pl.BlockSpec((blk, D), lambda i: (i, 0)),
            pl.BlockSpec((7, D, D), lambda i: (0, 0, 0)),
            pl.BlockSpec((7, 1, D), lambda i: (0, 0, 0)),
        ],
        out_specs=pl.BlockSpec((6, blk, D), lambda i: (0, i, 0)),
        out_shape=jax.ShapeDtypeStruct((6, N, D), jnp.float32),
    )(h, p, W, B)


def _edge_mm_body(e_ref, w_ref, b_ref, out_ref):
    out_ref[...] = jax.lax.dot_general(
        e_ref[...], w_ref[...], (((1,), (0,)), ((), ())),
        preferred_element_type=jnp.float32) + b_ref[...]


def _edge_matmul(e, W, b):
    blk = 4000
    return pl.pallas_call(
        _edge_mm_body,
        grid=(E // blk,),
        in_specs=[
            pl.BlockSpec((blk, D), lambda i: (i, 0)),
            pl.BlockSpec((D, D), lambda i: (0, 0)),
            pl.BlockSpec((1, D), lambda i: (0, 0)),
        ],
        out_specs=pl.BlockSpec((blk, D), lambda i: (i, 0)),
        out_shape=jax.ShapeDtypeStruct((E, D), jnp.float32),
    )(e, W, b.reshape(1, D))


def _clamp_body(d_ref, o0_ref, o1_ref):
    d = d_ref[...]
    scratch = NH + jnp.bitwise_and(d, 63)
    o0_ref[...] = jnp.where(d < NH, d, scratch)
    o1_ref[...] = jnp.where(d >= NH, d - NH, scratch)


def _clamp_dst(dst):
    d2 = dst.reshape(E // 128, 128)
    blk = E // 128
    assert (E // 128) % blk == 0
    o0, o1 = pl.pallas_call(
        _clamp_body,
        grid=(E // 128 // blk,),
        in_specs=[pl.BlockSpec((blk, 128), lambda i: (i, 0))],
        out_specs=[pl.BlockSpec((blk, 128), lambda i: (i, 0))] * 2,
        out_shape=[jax.ShapeDtypeStruct((E // 128, 128), jnp.int32)] * 2,
    )(d2)
    return o0.reshape(E), o1.reshape(E)


def _estats_body(x_ref, out_ref):
    i = pl.program_id(0)

    @pl.when(i == 0)
    def _():
        out_ref[...] = jnp.zeros_like(out_ref)

    x = x_ref[...]
    s = jnp.sum(x, axis=0, keepdims=True)
    s2 = jnp.sum(x * x, axis=0, keepdims=True)
    out_ref[...] += jnp.concatenate([s, s2], axis=0)


def _edge_stats(hat):
    blk = 8000
    return pl.pallas_call(
        _estats_body,
        grid=(E // blk,),
        in_specs=[pl.BlockSpec((blk, D), lambda i: (i, 0))],
        out_specs=pl.BlockSpec((2, D), lambda i: (0, 0)),
        out_shape=jax.ShapeDtypeStruct((2, D), jnp.float32),
    )(hat)


def _eapply_body(x_ref, st_ref, g_ref, b_ref, out_ref):
    m = st_ref[0:1] * (1.0 / E)
    var = st_ref[1:2] * (1.0 / E) - m * m
    inv = jax.lax.rsqrt(var + 1e-5)
    out_ref[...] = jnp.maximum((x_ref[...] - m) * inv * g_ref[...] + b_ref[...], 0.0)


def _edge_apply(hat, stats, g, b):
    blk = 8000
    return pl.pallas_call(
        _eapply_body,
        grid=(E // blk,),
        in_specs=[
            pl.BlockSpec((blk, D), lambda i: (i, 0)),
            pl.BlockSpec((2, D), lambda i: (0, 0)),
            pl.BlockSpec((1, D), lambda i: (0, 0)),
            pl.BlockSpec((1, D), lambda i: (0, 0)),
        ],
        out_specs=pl.BlockSpec((blk, D), lambda i: (i, 0)),
        out_shape=jax.ShapeDtypeStruct((E, D), jnp.float32),
    )(hat, stats, g.reshape(1, D), b.reshape(1, D))


def _combine_body(p_ref, out_ref):
    j = pl.program_id(0)
    out_ref[...] = p_ref[0, 0] + p_ref[1, 0]


def _combine(parts):
    return pl.pallas_call(
        _combine_body,
        grid=(2,),
        in_specs=[pl.BlockSpec((NC, 1, NH, D), lambda j: (0, j, 0, 0))],
        out_specs=pl.BlockSpec((NH, D), lambda j: (j, 0)),
        out_shape=jax.ShapeDtypeStruct((N, D), jnp.float32),
    )(parts)


def _nodefin_body(a1_ref, c1_ref, s_ref, sv_ref, sp_ref, g_ref, b_ref,
                  h_ref, p_ref):
    denom = s_ref[...] + 1e-6
    h_pre = a1_ref[...] + sv_ref[...] / denom
    p_pre = c1_ref[...] + sp_ref[...] / denom
    m = jnp.mean(h_pre, axis=0, keepdims=True)
    var = jnp.mean(h_pre * h_pre, axis=0, keepdims=True) - m * m
    inv = jax.lax.rsqrt(var + 1e-5)
    h_ref[...] = jnp.maximum((h_pre - m) * inv * g_ref[...] + b_ref[...], 0.0)
    p_ref[...] = jnp.tanh(p_pre)


def _node_finalize(A1_h, C1p, S, SV, SP, g, b):
    return pl.pallas_call(
        _nodefin_body,
        in_specs=[pl.BlockSpec((N, D), lambda: (0, 0))] * 5
        + [pl.BlockSpec((1, D), lambda: (0, 0))] * 2,
        out_specs=[pl.BlockSpec((N, D), lambda: (0, 0))] * 2,
        out_shape=[jax.ShapeDtypeStruct((N, D), jnp.float32)] * 2,
    )(A1_h, C1p, S, SV, SP, g.reshape(1, D), b.reshape(1, D))


_RCH = 80
_RPT = PAD // NS

CH = 80
EPW = E // NW
NCH = EPW // CH


def _zero_shared(acc, zbuf, tid):
    zb = zbuf.shape[0]

    @pl.loop(0, zb)
    def _(r):
        for v in range(NV):
            zbuf[r, pl.ds(v * 16, 16)] = jnp.zeros((16,), jnp.float32)

    @pl.loop(0, _RPT // _RCH)
    def _(j):
        pltpu.sync_copy(zbuf, acc.at[pl.ds(tid * _RPT + j * _RCH, _RCH)])


def _writeout_shared(acc, out_slot, tid):
    @pl.loop(0, _RPT // _RCH)
    def _(j):
        row0 = tid * _RPT + j * _RCH
        pltpu.sync_copy(acc.at[pl.ds(row0, _RCH)], out_slot.at[pl.ds(row0, _RCH)])


def _sc_edges_body(b3e_hbm, src_hbm, dst_hbm, d0_hbm, d1_hbm,
                   b1_hbm, b2_hbm, v_hbm, cp_hbm,
                   hat_hbm, sig_hbm, sigp_hbm, svp_hbm, spp_hbm,
                   srcb, dstb, scatb, xb, g1, g2, acc, zbuf, sem):
    c = jax.lax.axis_index("c")
    s = jax.lax.axis_index("s")
    wid = c * NS + s

    def writeout(out_hbm, half):
        @pl.when(c == 0)
        def _():
            _writeout_shared(acc, out_hbm.at[0].at[half], s)

        @pl.when(c == 1)
        def _():
            _writeout_shared(acc, out_hbm.at[1].at[half], s)

    dhbm = (d0_hbm, d1_hbm)

    def hat_phase(half):
        _zero_shared(acc, zbuf, s)
        plsc.subcore_barrier()

        @pl.loop(0, NCH)
        def _(k):
            base = wid * EPW + k * CH
            pltpu.sync_copy(src_hbm.at[pl.ds(base, CH)], srcb)
            pltpu.sync_copy(dst_hbm.at[pl.ds(base, CH)], dstb)
            pltpu.sync_copy(dhbm[half].at[pl.ds(base, CH)], scatb)
            cp0 = pltpu.async_copy(b3e_hbm.at[pl.ds(base, CH)], xb, sem)
            cp1 = pltpu.async_copy(b1_hbm.at[srcb], g1, sem)
            cp2 = pltpu.async_copy(b2_hbm.at[dstb], g2, sem)
            cp0.wait()
            cp1.wait()
            cp2.wait()

            @pl.loop(0, CH)
            def _(r):
                for v in range(NV):
                    sl = pl.ds(v * 16, 16)
                    x = xb[r, sl] + g1[r, sl] + g2[r, sl]
                    xb[r, sl] = x
                    g1[r, sl] = 1.0 / (1.0 + jnp.exp(-x))

            pltpu.sync_copy(xb, hat_hbm.at[pl.ds(base, CH)])
            pltpu.sync_copy(g1, sig_hbm.at[pl.ds(base, CH)])
            pltpu.sync_copy(g1, acc.at[scatb], add=True)

        plsc.subcore_barrier()
        writeout(sigp_hbm, half)
        plsc.subcore_barrier()

    def sig_phase(half):
        _zero_shared(acc, zbuf, s)
        plsc.subcore_barrier()

        @pl.loop(0, NCH)
        def _(k):
            base = wid * EPW + k * CH
            pltpu.sync_copy(dhbm[half].at[pl.ds(base, CH)], scatb)
            pltpu.async_copy(sig_hbm.at[pl.ds(base, CH)], g1, sem).wait()
            pltpu.sync_copy(g1, acc.at[scatb], add=True)

        plsc.subcore_barrier()
        writeout(sigp_hbm, half)
        plsc.subcore_barrier()

    def gate_phase(tab_hbm, out_hbm, half):
        _zero_shared(acc, zbuf, s)
        plsc.subcore_barrier()

        @pl.loop(0, NCH)
        def _(k):
            base = wid * EPW + k * CH
            pltpu.sync_copy(src_hbm.at[pl.ds(base, CH)], srcb)
            pltpu.sync_copy(dhbm[half].at[pl.ds(base, CH)], scatb)
            cp0 = pltpu.async_copy(sig_hbm.at[pl.ds(base, CH)], xb, sem)
            cp1 = pltpu.async_copy(tab_hbm.at[srcb], g1, sem)
            cp0.wait()
            cp1.wait()

            @pl.loop(0, CH)
            def _(r):
                for v in range(NV):
                    sl = pl.ds(v * 16, 16)
                    g1[r, sl] = g1[r, sl] * xb[r, sl]

            pltpu.sync_copy(g1, acc.at[scatb], add=True)

        plsc.subcore_barrier()
        writeout(out_hbm, half)
        plsc.subcore_barrier()

    hat_phase(0)
    sig_phase(1)
    for half in (0, 1):
        gate_phase(v_hbm, svp_hbm, half)
        gate_phase(cp_hbm, spp_hbm, half)


def _sc_edges(b3e, src, dst, d0, d1, b1n, b2n, vtab, cptab):
    f = pl.kernel(
        _sc_edges_body,
        out_type=[jax.ShapeDtypeStruct((E, D), jnp.float32),
                  jax.ShapeDtypeStruct((E, D), jnp.float32),
                  jax.ShapeDtypeStruct((NC, 2, PAD, D), jnp.float32),
                  jax.ShapeDtypeStruct((NC, 2, PAD, D), jnp.float32),
                  jax.ShapeDtypeStruct((NC, 2, PAD, D), jnp.float32)],
        mesh=_MESH,
        scratch_types=[
            pltpu.VMEM((CH,), jnp.int32),
            pltpu.VMEM((CH,), jnp.int32),
            pltpu.VMEM((CH,), jnp.int32),
            pltpu.VMEM((CH, D), jnp.float32),
            pltpu.VMEM((CH, D), jnp.float32),
            pltpu.VMEM((CH, D), jnp.float32),
            pltpu.VMEM_SHARED((PAD, D), jnp.float32),
            pltpu.VMEM((_RCH, D), jnp.float32),
            pltpu.SemaphoreType.DMA,
        ],
    )
    return f(b3e, src, dst, d0, d1, b1n, b2n, vtab, cptab)


@jax.jit
def kernel(h, e, p, edge_index, A1_W, A1_b, A2_W, A2_b, B1_W, B1_b, B2_W, B2_b,
           B3_W, B3_b, C1_W, C1_b, C2_W, C2_b, bn_h_g, bn_h_b, bn_e_g, bn_e_b):
    src = edge_index[0]
    dst = edge_index[1]
    d0, d1 = _clamp_dst(dst)

    W = jnp.stack([A1_W, B1_W, B2_W, C1_W, A2_W[:D], A2_W[D:], C2_W])
    B = jnp.stack([A1_b, B1_b, B2_b, C1_b, A2_b, jnp.zeros_like(A2_b), C2_b])[:, None, :]

    nodes = _node_precompute(h, p, W, B)
    B3e = _edge_matmul(e, B3_W, B3_b)

    hat, _sig, sigp, svp, spp = _sc_edges(
        B3e, src, dst, d0, d1, nodes[1], nodes[2], nodes[4], nodes[5])

    stats = _edge_stats(hat)
    e_new = _edge_apply(hat, stats, bn_e_g, bn_e_b)
    S = _combine(sigp)
    SV = _combine(svp)
    SP = _combine(spp)
    h_new, p_new = _node_finalize(nodes[0], nodes[3], S, SV, SP, bn_h_g, bn_h_b)
    return (h_new, e_new, p_new)


# speedup vs baseline: 2.7602x; 1.5280x over previous
import functools

import jax
import jax.numpy as jnp
from jax.experimental import pallas as pl
from jax.experimental.pallas import tpu as pltpu
from jax.experimental.pallas import tpu_sc as plsc

N, E, D = 10000, 320000, 128
NC, NS = 2, 16
NW = NC * NS
NV = D // 16
NH = N // 2
PAD = 5120

_MESH = plsc.VectorSubcoreMesh(core_axis_name="c", subcore_axis_name="s")


def _node_pre_body(h_ref, p_ref, w_ref, b_ref, out_ref):
    h = h_ref[...]
    p = p_ref[...]

    def mm(x, i):
        return jax.lax.dot_general(
            x, w_ref[i], (((1,), (0,)), ((), ())),
            preferred_element_type=jnp.float32) + b_ref[i]

    out_ref[0] = mm(h, 0)
    out_ref[1] = mm(h, 1)
    out_ref[2] = mm(h, 2)
    out_ref[3] = mm(p, 3)
    out_ref[4] = mm(h, 4) + jax.lax.dot_general(
        p, w_ref[5], (((1,), (0,)), ((), ())),
        preferred_element_type=jnp.float32)
    out_ref[5] = mm(p, 6)


def _node_precompute(h, p, W, B):
    blk = 2000
    return pl.pallas_call(
        _node_pre_body,
        grid=(N // blk,),
        in_specs=[
            pl.BlockSpec((blk, D), lambda i: (i, 0)),
            pl.BlockSpec((blk, D), lambda i: (i, 0)),
            pl.BlockSpec((7, D, D), lambda i: (0, 0, 0)),
            pl.BlockSpec((7, 1, D), lambda i: (0, 0, 0)),
        ],
        out_specs=pl.BlockSpec((6, blk, D), lambda i: (0, i, 0)),
        out_shape=jax.ShapeDtypeStruct((6, N, D), jnp.float32),
    )(h, p, W, B)


def _edge_mm_body(e_ref, w_ref, b_ref, out_ref):
    out_ref[...] = jax.lax.dot_general(
        e_ref[...], w_ref[...], (((1,), (0,)), ((), ())),
        preferred_element_type=jnp.float32) + b_ref[...]


def _edge_matmul(e, W, b):
    blk = 4000
    return pl.pallas_call(
        _edge_mm_body,
        grid=(E // blk,),
        in_specs=[
            pl.BlockSpec((blk, D), lambda i: (i, 0)),
            pl.BlockSpec((D, D), lambda i: (0, 0)),
            pl.BlockSpec((1, D), lambda i: (0, 0)),
        ],
        out_specs=pl.BlockSpec((blk, D), lambda i: (i, 0)),
        out_shape=jax.ShapeDtypeStruct((E, D), jnp.float32),
    )(e, W, b.reshape(1, D))


def _clamp_body(d_ref, o0_ref, o1_ref):
    d = d_ref[...]
    scratch = NH + jnp.bitwise_and(d, 63)
    o0_ref[...] = jnp.where(d < NH, d, scratch)
    o1_ref[...] = jnp.where(d >= NH, d - NH, scratch)


def _clamp_dst(dst):
    d2 = dst.reshape(E // 128, 128)
    blk = E // 128
    assert (E // 128) % blk == 0
    o0, o1 = pl.pallas_call(
        _clamp_body,
        grid=(E // 128 // blk,),
        in_specs=[pl.BlockSpec((blk, 128), lambda i: (i, 0))],
        out_specs=[pl.BlockSpec((blk, 128), lambda i: (i, 0))] * 2,
        out_shape=[jax.ShapeDtypeStruct((E // 128, 128), jnp.int32)] * 2,
    )(d2)
    return o0.reshape(E), o1.reshape(E)


def _estats_body(x_ref, out_ref):
    i = pl.program_id(0)

    @pl.when(i == 0)
    def _():
        out_ref[...] = jnp.zeros_like(out_ref)

    x = x_ref[...]
    s = jnp.sum(x, axis=0, keepdims=True)
    s2 = jnp.sum(x * x, axis=0, keepdims=True)
    out_ref[...] += jnp.concatenate([s, s2], axis=0)


def _edge_stats(hat):
    blk = 8000
    return pl.pallas_call(
        _estats_body,
        grid=(E // blk,),
        in_specs=[pl.BlockSpec((blk, D), lambda i: (i, 0))],
        out_specs=pl.BlockSpec((2, D), lambda i: (0, 0)),
        out_shape=jax.ShapeDtypeStruct((2, D), jnp.float32),
    )(hat)


def _eapply_body(x_ref, st_ref, g_ref, b_ref, out_ref):
    m = st_ref[0:1] * (1.0 / E)
    var = st_ref[1:2] * (1.0 / E) - m * m
    inv = jax.lax.rsqrt(var + 1e-5)
    out_ref[...] = jnp.maximum((x_ref[...] - m) * inv * g_ref[...] + b_ref[...], 0.0)


def _edge_apply(hat, stats, g, b):
    blk = 8000
    return pl.pallas_call(
        _eapply_body,
        grid=(E // blk,),
        in_specs=[
            pl.BlockSpec((blk, D), lambda i: (i, 0)),
            pl.BlockSpec((2, D), lambda i: (0, 0)),
            pl.BlockSpec((1, D), lambda i: (0, 0)),
            pl.BlockSpec((1, D), lambda i: (0, 0)),
        ],
        out_specs=pl.BlockSpec((blk, D), lambda i: (i, 0)),
        out_shape=jax.ShapeDtypeStruct((E, D), jnp.float32),
    )(hat, stats, g.reshape(1, D), b.reshape(1, D))


def _combine_body(p_ref, out_ref):
    j = pl.program_id(0)
    out_ref[...] = p_ref[0, 0] + p_ref[1, 0]


def _combine(parts):
    return pl.pallas_call(
        _combine_body,
        grid=(2,),
        in_specs=[pl.BlockSpec((NC, 1, NH, D), lambda j: (0, j, 0, 0))],
        out_specs=pl.BlockSpec((NH, D), lambda j: (j, 0)),
        out_shape=jax.ShapeDtypeStruct((N, D), jnp.float32),
    )(parts)


def _nodefin_body(a1_ref, c1_ref, s_ref, sv_ref, sp_ref, g_ref, b_ref,
                  h_ref, p_ref):
    denom = s_ref[...] + 1e-6
    h_pre = a1_ref[...] + sv_ref[...] / denom
    p_pre = c1_ref[...] + sp_ref[...] / denom
    m = jnp.mean(h_pre, axis=0, keepdims=True)
    var = jnp.mean(h_pre * h_pre, axis=0, keepdims=True) - m * m
    inv = jax.lax.rsqrt(var + 1e-5)
    h_ref[...] = jnp.maximum((h_pre - m) * inv * g_ref[...] + b_ref[...], 0.0)
    p_ref[...] = jnp.tanh(p_pre)


def _node_finalize(A1_h, C1p, S, SV, SP, g, b):
    return pl.pallas_call(
        _nodefin_body,
        in_specs=[pl.BlockSpec((N, D), lambda: (0, 0))] * 5
        + [pl.BlockSpec((1, D), lambda: (0, 0))] * 2,
        out_specs=[pl.BlockSpec((N, D), lambda: (0, 0))] * 2,
        out_shape=[jax.ShapeDtypeStruct((N, D), jnp.float32)] * 2,
    )(A1_h, C1p, S, SV, SP, g.reshape(1, D), b.reshape(1, D))


_RCH = 80
_RPT = PAD // NS

CH = 80
EPW = E // NW
NCH = EPW // CH


_ZROWS = 16


def _zero_shared(acc, zbuf, tid):
    @pl.loop(0, _ZROWS)
    def _(r):
        for v in range(NV):
            zbuf[r, pl.ds(v * 16, 16)] = jnp.zeros((16,), jnp.float32)

    @pl.loop(0, _RPT // _ZROWS)
    def _(j):
        pltpu.sync_copy(zbuf, acc.at[pl.ds(tid * _RPT + j * _ZROWS, _ZROWS)])


def _writeout_shared(acc, out_slot, tid):
    @pl.loop(0, _RPT // _RCH)
    def _(j):
        row0 = tid * _RPT + j * _RCH
        pltpu.sync_copy(acc.at[pl.ds(row0, _RCH)], out_slot.at[pl.ds(row0, _RCH)])


_NBUF = 2


def _sc_edges_body(b3e_hbm, src_hbm, dst_hbm, d0_hbm, d1_hbm,
                   b1_hbm, b2_hbm, v_hbm, cp_hbm,
                   hat_hbm, sig_hbm, sigp_hbm, svp_hbm, spp_hbm,
                   sb0, sb1, db0, db1, cb0, cb1,
                   xb0, xb1, g10, g11, g20, g21,
                   acc, zbuf,
                   si0, si1, so0, so1):
    c = jax.lax.axis_index("c")
    s = jax.lax.axis_index("s")
    wid = c * NS + s
    srcb = (sb0, sb1)
    dstb = (db0, db1)
    scatb = (cb0, cb1)
    xb = (xb0, xb1)
    g1 = (g10, g11)
    g2 = (g20, g21)
    semi = (si0, si1)
    semo = (so0, so1)
    dhbm = (d0_hbm, d1_hbm)

    def writeout(out_hbm, half):
        @pl.when(c == 0)
        def _():
            _writeout_shared(acc, out_hbm.at[0].at[half], s)

        @pl.when(c == 1)
        def _():
            _writeout_shared(acc, out_hbm.at[1].at[half], s)

    def run_phase(issue_in, wait_in, compute, issue_out, drain_out):
        issue_in(0, 0)

        @pl.loop(0, NCH + (-NCH) % _NBUF, step=_NBUF)
        def _(k0):
            for i in range(_NBUF):
                b = i
                k = k0 + i

                @pl.when(k < NCH)
                def _():
                    bn = 1 - b

                    @pl.when(k >= 1)
                    def _():
                        drain_out(bn, k - 1)

                    @pl.when(k + 1 < NCH)
                    def _():
                        issue_in(bn, k + 1)

                    wait_in(b, k)
                    compute(b)
                    issue_out(b, k)

        drain_out((NCH - 1) % _NBUF, NCH - 1)

    def hat_phase(half):
        _zero_shared(acc, zbuf, s)
        plsc.subcore_barrier()

        def issue_in(b, k):
            base = wid * EPW + k * CH
            pltpu.sync_copy(src_hbm.at[pl.ds(base, CH)], srcb[b])
            pltpu.sync_copy(dst_hbm.at[pl.ds(base, CH)], dstb[b])
            pltpu.sync_copy(dhbm[half].at[pl.ds(base, CH)], scatb[b])
            pltpu.async_copy(b3e_hbm.at[pl.ds(base, CH)], xb[b], semi[b])
            pltpu.async_copy(b1_hbm.at[srcb[b]], g1[b], semi[b])
            pltpu.async_copy(b2_hbm.at[dstb[b]], g2[b], semi[b])

        def wait_in(b, k):
            base = wid * EPW + k * CH
            pltpu.make_async_copy(b3e_hbm.at[pl.ds(base, CH)], xb[b], semi[b]).wait()
            pltpu.make_async_copy(b1_hbm.at[srcb[b]], g1[b], semi[b]).wait()
            pltpu.make_async_copy(b2_hbm.at[dstb[b]], g2[b], semi[b]).wait()

        def compute(b):
            xbb = xb[b]
            g1b = g1[b]
            g2b = g2[b]

            @pl.loop(0, CH)
            def _(r):
                for v in range(NV):
                    sl = pl.ds(v * 16, 16)
                    x = xbb[r, sl] + g1b[r, sl] + g2b[r, sl]
                    xbb[r, sl] = x
                    g1b[r, sl] = 1.0 / (1.0 + jnp.exp(-x))

        def issue_out(b, k):
            base = wid * EPW + k * CH
            pltpu.async_copy(xb[b], hat_hbm.at[pl.ds(base, CH)], semo[b])
            pltpu.async_copy(g1[b], sig_hbm.at[pl.ds(base, CH)], semo[b])
            pltpu.sync_copy(g1[b], acc.at[scatb[b]], add=True)

        def drain_out(b, k):
            base = wid * EPW + k * CH
            pltpu.make_async_copy(xb[b], hat_hbm.at[pl.ds(base, CH)], semo[b]).wait()
            pltpu.make_async_copy(g1[b], sig_hbm.at[pl.ds(base, CH)], semo[b]).wait()

        run_phase(issue_in, wait_in, compute, issue_out, drain_out)
        plsc.subcore_barrier()
        writeout(sigp_hbm, half)
        plsc.subcore_barrier()

    def sig_phase(half):
        _zero_shared(acc, zbuf, s)
        plsc.subcore_barrier()

        def issue_in(b, k):
            base = wid * EPW + k * CH
            pltpu.sync_copy(dhbm[half].at[pl.ds(base, CH)], scatb[b])
            pltpu.async_copy(sig_hbm.at[pl.ds(base, CH)], g1[b], semi[b])

        def wait_in(b, k):
            base = wid * EPW + k * CH
            pltpu.make_async_copy(sig_hbm.at[pl.ds(base, CH)], g1[b], semi[b]).wait()

        def compute(b):
            pass

        def issue_out(b, k):
            pltpu.sync_copy(g1[b], acc.at[scatb[b]], add=True)

        def drain_out(b, k):
            pass

        run_phase(issue_in, wait_in, compute, issue_out, drain_out)
        plsc.subcore_barrier()
        writeout(sigp_hbm, half)
        plsc.subcore_barrier()

    def gate_phase(tab_hbm, out_hbm, half):
        _zero_shared(acc, zbuf, s)
        plsc.subcore_barrier()

        def issue_in(b, k):
            base = wid * EPW + k * CH
            pltpu.sync_copy(src_hbm.at[pl.ds(base, CH)], srcb[b])
            pltpu.sync_copy(dhbm[half].at[pl.ds(base, CH)], scatb[b])
            pltpu.async_copy(sig_hbm.at[pl.ds(base, CH)], xb[b], semi[b])
            pltpu.async_copy(tab_hbm.at[srcb[b]], g1[b], semi[b])

        def wait_in(b, k):
            base = wid * EPW + k * CH
            pltpu.make_async_copy(sig_hbm.at[pl.ds(base, CH)], xb[b], semi[b]).wait()
            pltpu.make_async_copy(tab_hbm.at[srcb[b]], g1[b], semi[b]).wait()

        def compute(b):
            xbb = xb[b]
            g1b = g1[b]

            @pl.loop(0, CH)
            def _(r):
                for v in range(NV):
                    sl = pl.ds(v * 16, 16)
                    g1b[r, sl] = g1b[r, sl] * xbb[r, sl]

        def issue_out(b, k):
            pltpu.sync_copy(g1[b], acc.at[scatb[b]], add=True)

        def drain_out(b, k):
            pass

        run_phase(issue_in, wait_in, compute, issue_out, drain_out)
        plsc.subcore_barrier()
        writeout(out_hbm, half)
        plsc.subcore_barrier()

    hat_phase(0)
    gate_phase(v_hbm, svp_hbm, 0)
    gate_phase(cp_hbm, spp_hbm, 0)
    sig_phase(1)
    gate_phase(v_hbm, svp_hbm, 1)
    gate_phase(cp_hbm, spp_hbm, 1)


def _sc_edges(b3e, src, dst, d0, d1, b1n, b2n, vtab, cptab):
    f = pl.kernel(
        _sc_edges_body,
        out_type=[jax.ShapeDtypeStruct((E, D), jnp.float32),
                  jax.ShapeDtypeStruct((E, D), jnp.float32),
                  jax.ShapeDtypeStruct((NC, 2, PAD, D), jnp.float32),
                  jax.ShapeDtypeStruct((NC, 2, PAD, D), jnp.float32),
                  jax.ShapeDtypeStruct((NC, 2, PAD, D), jnp.float32)],
        mesh=_MESH,
        scratch_types=[pltpu.VMEM((CH,), jnp.int32)] * 6
        + [pltpu.VMEM((CH, D), jnp.float32)] * 6
        + [
            pltpu.VMEM_SHARED((PAD, D), jnp.float32),
            pltpu.VMEM((_ZROWS, D), jnp.float32),
        ]
        + [pltpu.SemaphoreType.DMA] * 4,
    )
    return f(b3e, src, dst, d0, d1, b1n, b2n, vtab, cptab)


@jax.jit
def kernel(h, e, p, edge_index, A1_W, A1_b, A2_W, A2_b, B1_W, B1_b, B2_W, B2_b,
           B3_W, B3_b, C1_W, C1_b, C2_W, C2_b, bn_h_g, bn_h_b, bn_e_g, bn_e_b):
    src = edge_index[0]
    dst = edge_index[1]
    d0, d1 = _clamp_dst(dst)

    W = jnp.stack([A1_W, B1_W, B2_W, C1_W, A2_W[:D], A2_W[D:], C2_W])
    B = jnp.stack([A1_b, B1_b, B2_b, C1_b, A2_b, jnp.zeros_like(A2_b), C2_b])[:, None, :]

    nodes = _node_precompute(h, p, W, B)
    B3e = _edge_matmul(e, B3_W, B3_b)

    hat, _sig, sigp, svp, spp = _sc_edges(
        B3e, src, dst, d0, d1, nodes[1], nodes[2], nodes[4], nodes[5])

    stats = _edge_stats(hat)
    e_new = _edge_apply(hat, stats, bn_e_g, bn_e_b)
    S = _combine(sigp)
    SV = _combine(svp)
    SP = _combine(spp)
    h_new, p_new = _node_finalize(nodes[0], nodes[3], S, SV, SP, bn_h_g, bn_h_b)
    return (h_new, e_new, p_new)


# speedup vs baseline: 2.8184x; 1.0211x over previous
import functools

import jax
import jax.numpy as jnp
from jax.experimental import pallas as pl
from jax.experimental.pallas import tpu as pltpu
from jax.experimental.pallas import tpu_sc as plsc

N, E, D = 10000, 320000, 128
NC, NS = 2, 16
NW = NC * NS
NV = D // 16
NH = N // 2
PAD = 5120

_MESH = plsc.VectorSubcoreMesh(core_axis_name="c", subcore_axis_name="s")


def _node_pre_body(h_ref, p_ref, w_ref, b_ref, out_ref):
    h = h_ref[...]
    p = p_ref[...]

    def mm(x, i):
        return jax.lax.dot_general(
            x, w_ref[i], (((1,), (0,)), ((), ())),
            preferred_element_type=jnp.float32) + b_ref[i]

    out_ref[0] = mm(h, 0)
    out_ref[1] = mm(h, 1)
    out_ref[2] = mm(h, 2)
    out_ref[3] = mm(p, 3)
    out_ref[4] = mm(h, 4) + jax.lax.dot_general(
        p, w_ref[5], (((1,), (0,)), ((), ())),
        preferred_element_type=jnp.float32)
    out_ref[5] = mm(p, 6)


def _node_precompute(h, p, W, B):
    blk = 2000
    return pl.pallas_call(
        _node_pre_body,
        grid=(N // blk,),
        in_specs=[
            pl.BlockSpec((blk, D), lambda i: (i, 0)),
            pl.BlockSpec((blk, D), lambda i: (i, 0)),
            pl.BlockSpec((7, D, D), lambda i: (0, 0, 0)),
            pl.BlockSpec((7, 1, D), lambda i: (0, 0, 0)),
        ],
        out_specs=pl.BlockSpec((6, blk, D), lambda i: (0, i, 0)),
        out_shape=jax.ShapeDtypeStruct((6, N, D), jnp.float32),
    )(h, p, W, B)


def _edge_mm_body(e_ref, w_ref, b_ref, out_ref):
    out_ref[...] = jax.lax.dot_general(
        e_ref[...], w_ref[...], (((1,), (0,)), ((), ())),
        preferred_element_type=jnp.float32) + b_ref[...]


def _edge_matmul(e, W, b):
    blk = 4000
    return pl.pallas_call(
        _edge_mm_body,
        grid=(E // blk,),
        in_specs=[
            pl.BlockSpec((blk, D), lambda i: (i, 0)),
            pl.BlockSpec((D, D), lambda i: (0, 0)),
            pl.BlockSpec((1, D), lambda i: (0, 0)),
        ],
        out_specs=pl.BlockSpec((blk, D), lambda i: (i, 0)),
        out_shape=jax.ShapeDtypeStruct((E, D), jnp.float32),
    )(e, W, b.reshape(1, D))


def _clamp_body(d_ref, o0_ref, o1_ref):
    d = d_ref[...]
    scratch = NH + jnp.bitwise_and(d, 63)
    o0_ref[...] = jnp.where(d < NH, d, scratch)
    o1_ref[...] = jnp.where(d >= NH, d - NH, scratch)


def _clamp_dst(dst):
    d2 = dst.reshape(E // 128, 128)
    blk = E // 128
    assert (E // 128) % blk == 0
    o0, o1 = pl.pallas_call(
        _clamp_body,
        grid=(E // 128 // blk,),
        in_specs=[pl.BlockSpec((blk, 128), lambda i: (i, 0))],
        out_specs=[pl.BlockSpec((blk, 128), lambda i: (i, 0))] * 2,
        out_shape=[jax.ShapeDtypeStruct((E // 128, 128), jnp.int32)] * 2,
    )(d2)
    return o0.reshape(E), o1.reshape(E)


def _estats_body(x_ref, out_ref):
    i = pl.program_id(0)

    @pl.when(i == 0)
    def _():
        out_ref[...] = jnp.zeros_like(out_ref)

    x = x_ref[...]
    s = jnp.sum(x, axis=0, keepdims=True)
    s2 = jnp.sum(x * x, axis=0, keepdims=True)
    out_ref[...] += jnp.concatenate([s, s2], axis=0)


def _edge_stats(hat):
    blk = 8000
    return pl.pallas_call(
        _estats_body,
        grid=(E // blk,),
        in_specs=[pl.BlockSpec((blk, D), lambda i: (i, 0))],
        out_specs=pl.BlockSpec((2, D), lambda i: (0, 0)),
        out_shape=jax.ShapeDtypeStruct((2, D), jnp.float32),
    )(hat)


def _eapply_body(x_ref, st_ref, g_ref, b_ref, out_ref):
    m = st_ref[0:1] * (1.0 / E)
    var = st_ref[1:2] * (1.0 / E) - m * m
    inv = jax.lax.rsqrt(var + 1e-5)
    out_ref[...] = jnp.maximum((x_ref[...] - m) * inv * g_ref[...] + b_ref[...], 0.0)


def _edge_apply(hat, stats, g, b):
    blk = 8000
    return pl.pallas_call(
        _eapply_body,
        grid=(E // blk,),
        in_specs=[
            pl.BlockSpec((blk, D), lambda i: (i, 0)),
            pl.BlockSpec((2, D), lambda i: (0, 0)),
            pl.BlockSpec((1, D), lambda i: (0, 0)),
            pl.BlockSpec((1, D), lambda i: (0, 0)),
        ],
        out_specs=pl.BlockSpec((blk, D), lambda i: (i, 0)),
        out_shape=jax.ShapeDtypeStruct((E, D), jnp.float32),
    )(hat, stats, g.reshape(1, D), b.reshape(1, D))


def _combine_body(p_ref, out_ref):
    j = pl.program_id(0)
    out_ref[...] = p_ref[0, 0] + p_ref[1, 0]


def _combine(parts):
    return pl.pallas_call(
        _combine_body,
        grid=(2,),
        in_specs=[pl.BlockSpec((NC, 1, NH, D), lambda j: (0, j, 0, 0))],
        out_specs=pl.BlockSpec((NH, D), lambda j: (j, 0)),
        out_shape=jax.ShapeDtypeStruct((N, D), jnp.float32),
    )(parts)


def _nodefin_body(a1_ref, c1_ref, s_ref, sv_ref, sp_ref, g_ref, b_ref,
                  h_ref, p_ref):
    denom = s_ref[...] + 1e-6
    h_pre = a1_ref[...] + sv_ref[...] / denom
    p_pre = c1_ref[...] + sp_ref[...] / denom
    m = jnp.mean(h_pre, axis=0, keepdims=True)
    var = jnp.mean(h_pre * h_pre, axis=0, keepdims=True) - m * m
    inv = jax.lax.rsqrt(var + 1e-5)
    h_ref[...] = jnp.maximum((h_pre - m) * inv * g_ref[...] + b_ref[...], 0.0)
    p_ref[...] = jnp.tanh(p_pre)


def _node_finalize(A1_h, C1p, S, SV, SP, g, b):
    return pl.pallas_call(
        _nodefin_body,
        in_specs=[pl.BlockSpec((N, D), lambda: (0, 0))] * 5
        + [pl.BlockSpec((1, D), lambda: (0, 0))] * 2,
        out_specs=[pl.BlockSpec((N, D), lambda: (0, 0))] * 2,
        out_shape=[jax.ShapeDtypeStruct((N, D), jnp.float32)] * 2,
    )(A1_h, C1p, S, SV, SP, g.reshape(1, D), b.reshape(1, D))


_RCH = 80
_RPT = PAD // NS

CH = 80
EPW = E // NW
NCH = EPW // CH


_ZROWS = 16


def _zero_shared(acc, zbuf, tid):
    @pl.loop(0, _ZROWS)
    def _(r):
        for v in range(NV):
            zbuf[r, pl.ds(v * 16, 16)] = jnp.zeros((16,), jnp.float32)

    @pl.loop(0, _RPT // _ZROWS)
    def _(j):
        pltpu.sync_copy(zbuf, acc.at[pl.ds(tid * _RPT + j * _ZROWS, _ZROWS)])


def _writeout_shared(acc, out_slot, tid):
    @pl.loop(0, _RPT // _RCH)
    def _(j):
        row0 = tid * _RPT + j * _RCH
        pltpu.sync_copy(acc.at[pl.ds(row0, _RCH)], out_slot.at[pl.ds(row0, _RCH)])


_NBUF = 2


def _sc_edges_body(b3e_hbm, src_hbm, dst_hbm, d0_hbm, d1_hbm,
                   b1_hbm, b2_hbm, v_hbm, cp_hbm,
                   hat_hbm, sig_hbm, sigp_hbm, svp_hbm, spp_hbm,
                   sb0, sb1, db0, db1, cb0, cb1,
                   xb0, xb1, g10, g11, g20, g21,
                   acc, zbuf,
                   si0, si1, so0, so1):
    c = jax.lax.axis_index("c")
    s = jax.lax.axis_index("s")
    wid = c * NS + s
    srcb = (sb0, sb1)
    dstb = (db0, db1)
    scatb = (cb0, cb1)
    xb = (xb0, xb1)
    g1 = (g10, g11)
    g2 = (g20, g21)
    semi = (si0, si1)
    semo = (so0, so1)
    dhbm = (d0_hbm, d1_hbm)

    def writeout(out_hbm, half):
        @pl.when(c == 0)
        def _():
            _writeout_shared(acc, out_hbm.at[0].at[half], s)

        @pl.when(c == 1)
        def _():
            _writeout_shared(acc, out_hbm.at[1].at[half], s)

    def run_phase(issue_in, wait_in, compute, issue_out, drain_out):
        issue_in(0, 0)

        @pl.loop(0, NCH + (-NCH) % _NBUF, step=_NBUF)
        def _(k0):
            for i in range(_NBUF):
                b = i
                k = k0 + i

                @pl.when(k < NCH)
                def _():
                    bn = 1 - b

                    @pl.when(k >= 1)
                    def _():
                        drain_out(bn, k - 1)

                    @pl.when(k + 1 < NCH)
                    def _():
                        issue_in(bn, k + 1)

                    wait_in(b, k)
                    compute(b)
                    issue_out(b, k)

        drain_out((NCH - 1) % _NBUF, NCH - 1)

    def hat_phase(half):
        _zero_shared(acc, zbuf, s)
        plsc.subcore_barrier()

        def issue_in(b, k):
            base = wid * EPW + k * CH
            pltpu.sync_copy(src_hbm.at[pl.ds(base, CH)], srcb[b])
            pltpu.sync_copy(dst_hbm.at[pl.ds(base, CH)], dstb[b])
            pltpu.sync_copy(dhbm[half].at[pl.ds(base, CH)], scatb[b])
            pltpu.async_copy(b3e_hbm.at[pl.ds(base, CH)], xb[b], semi[b])
            pltpu.async_copy(b1_hbm.at[srcb[b]], g1[b], semi[b])
            pltpu.async_copy(b2_hbm.at[dstb[b]], g2[b], semi[b])

        def wait_in(b, k):
            base = wid * EPW + k * CH
            pltpu.make_async_copy(b3e_hbm.at[pl.ds(base, CH)], xb[b], semi[b]).wait()
            pltpu.make_async_copy(b1_hbm.at[srcb[b]], g1[b], semi[b]).wait()
            pltpu.make_async_copy(b2_hbm.at[dstb[b]], g2[b], semi[b]).wait()

        def compute(b):
            xbb = xb[b]
            g1b = g1[b]
            g2b = g2[b]

            @pl.loop(0, CH, step=2)
            def _(r0):
                for dr in range(2):
                    r = r0 + dr
                    for v in range(NV):
                        sl = pl.ds(v * 16, 16)
                        x = xbb[r, sl] + g1b[r, sl] + g2b[r, sl]
                        xbb[r, sl] = x
                        g1b[r, sl] = 1.0 / (1.0 + jnp.exp(-x))

        def issue_out(b, k):
            base = wid * EPW + k * CH
            pltpu.async_copy(xb[b], hat_hbm.at[pl.ds(base, CH)], semo[b])
            pltpu.async_copy(g1[b], sig_hbm.at[pl.ds(base, CH)], semo[b])
            pltpu.sync_copy(g1[b], acc.at[scatb[b]], add=True)

        def drain_out(b, k):
            base = wid * EPW + k * CH
            pltpu.make_async_copy(xb[b], hat_hbm.at[pl.ds(base, CH)], semo[b]).wait()
            pltpu.make_async_copy(g1[b], sig_hbm.at[pl.ds(base, CH)], semo[b]).wait()

        run_phase(issue_in, wait_in, compute, issue_out, drain_out)
        plsc.subcore_barrier()
        writeout(sigp_hbm, half)
        plsc.subcore_barrier()

    def sig_phase(half):
        _zero_shared(acc, zbuf, s)
        plsc.subcore_barrier()

        def issue_in(b, k):
            base = wid * EPW + k * CH
            pltpu.sync_copy(dhbm[half].at[pl.ds(base, CH)], scatb[b])
            pltpu.async_copy(sig_hbm.at[pl.ds(base, CH)], g1[b], semi[b])

        def wait_in(b, k):
            base = wid * EPW + k * CH
            pltpu.make_async_copy(sig_hbm.at[pl.ds(base, CH)], g1[b], semi[b]).wait()

        def compute(b):
            pass

        def issue_out(b, k):
            pltpu.sync_copy(g1[b], acc.at[scatb[b]], add=True)

        def drain_out(b, k):
            pass

        run_phase(issue_in, wait_in, compute, issue_out, drain_out)
        plsc.subcore_barrier()
        writeout(sigp_hbm, half)
        plsc.subcore_barrier()

    def gate_phase(tab_hbm, out_hbm, half):
        _zero_shared(acc, zbuf, s)
        plsc.subcore_barrier()

        def issue_in(b, k):
            base = wid * EPW + k * CH
            pltpu.sync_copy(src_hbm.at[pl.ds(base, CH)], srcb[b])
            pltpu.sync_copy(dhbm[half].at[pl.ds(base, CH)], scatb[b])
            pltpu.async_copy(sig_hbm.at[pl.ds(base, CH)], xb[b], semi[b])
            pltpu.async_copy(tab_hbm.at[srcb[b]], g1[b], semi[b])

        def wait_in(b, k):
            base = wid * EPW + k * CH
            pltpu.make_async_copy(sig_hbm.at[pl.ds(base, CH)], xb[b], semi[b]).wait()
            pltpu.make_async_copy(tab_hbm.at[srcb[b]], g1[b], semi[b]).wait()

        def compute(b):
            xbb = xb[b]
            g1b = g1[b]

            @pl.loop(0, CH, step=2)
            def _(r0):
                for dr in range(2):
                    r = r0 + dr
                    for v in range(NV):
                        sl = pl.ds(v * 16, 16)
                        g1b[r, sl] = g1b[r, sl] * xbb[r, sl]

        def issue_out(b, k):
            pltpu.sync_copy(g1[b], acc.at[scatb[b]], add=True)

        def drain_out(b, k):
            pass

        run_phase(issue_in, wait_in, compute, issue_out, drain_out)
        plsc.subcore_barrier()
        writeout(out_hbm, half)
        plsc.subcore_barrier()

    hat_phase(0)
    gate_phase(v_hbm, svp_hbm, 0)
    gate_phase(cp_hbm, spp_hbm, 0)
    sig_phase(1)
    gate_phase(v_hbm, svp_hbm, 1)
    gate_phase(cp_hbm, spp_hbm, 1)


def _sc_edges(b3e, src, dst, d0, d1, b1n, b2n, vtab, cptab):
    f = pl.kernel(
        _sc_edges_body,
        out_type=[jax.ShapeDtypeStruct((E, D), jnp.float32),
                  jax.ShapeDtypeStruct((E, D), jnp.float32),
                  jax.ShapeDtypeStruct((NC, 2, PAD, D), jnp.float32),
                  jax.ShapeDtypeStruct((NC, 2, PAD, D), jnp.float32),
                  jax.ShapeDtypeStruct((NC, 2, PAD, D), jnp.float32)],
        mesh=_MESH,
        scratch_types=[pltpu.VMEM((CH,), jnp.int32)] * 6
        + [pltpu.VMEM((CH, D), jnp.float32)] * 6
        + [
            pltpu.VMEM_SHARED((PAD, D), jnp.float32),
            pltpu.VMEM((_ZROWS, D), jnp.float32),
        ]
        + [pltpu.SemaphoreType.DMA] * 4,
    )
    return f(b3e, src, dst, d0, d1, b1n, b2n, vtab, cptab)


@jax.jit
def kernel(h, e, p, edge_index, A1_W, A1_b, A2_W, A2_b, B1_W, B1_b, B2_W, B2_b,
           B3_W, B3_b, C1_W, C1_b, C2_W, C2_b, bn_h_g, bn_h_b, bn_e_g, bn_e_b):
    src = edge_index[0]
    dst = edge_index[1]
    d0, d1 = _clamp_dst(dst)

    W = jnp.stack([A1_W, B1_W, B2_W, C1_W, A2_W[:D], A2_W[D:], C2_W])
    B = jnp.stack([A1_b, B1_b, B2_b, C1_b, A2_b, jnp.zeros_like(A2_b), C2_b])[:, None, :]

    nodes = _node_precompute(h, p, W, B)
    B3e = _edge_matmul(e, B3_W, B3_b)

    hat, _sig, sigp, svp, spp = _sc_edges(
        B3e, src, dst, d0, d1, nodes[1], nodes[2], nodes[4], nodes[5])

    stats = _edge_stats(hat)
    e_new = _edge_apply(hat, stats, bn_e_g, bn_e_b)
    S = _combine(sigp)
    SV = _combine(svp)
    SP = _combine(spp)
    h_new, p_new = _node_finalize(nodes[0], nodes[3], S, SV, SP, bn_h_g, bn_h_b)
    return (h_new, e_new, p_new)


# speedup vs baseline: 2.9191x; 1.0357x over previous
import functools

import jax
import jax.numpy as jnp
from jax.experimental import pallas as pl
from jax.experimental.pallas import tpu as pltpu
from jax.experimental.pallas import tpu_sc as plsc

N, E, D = 10000, 320000, 128
NC, NS = 2, 16
NW = NC * NS
NV = D // 16
NH = N // 2
PAD = 5120

_MESH = plsc.VectorSubcoreMesh(core_axis_name="c", subcore_axis_name="s")


def _node_pre_body(h_ref, p_ref, w_ref, b_ref, out_ref):
    h = h_ref[...]
    p = p_ref[...]

    def mm(x, i):
        return jax.lax.dot_general(
            x, w_ref[i], (((1,), (0,)), ((), ())),
            preferred_element_type=jnp.float32) + b_ref[i]

    out_ref[0] = mm(h, 0)
    out_ref[1] = mm(h, 1)
    out_ref[2] = mm(h, 2)
    out_ref[3] = mm(p, 3)
    out_ref[4] = mm(h, 4) + jax.lax.dot_general(
        p, w_ref[5], (((1,), (0,)), ((), ())),
        preferred_element_type=jnp.float32)
    out_ref[5] = mm(p, 6)


def _node_precompute(h, p, W, B):
    blk = 2000
    return pl.pallas_call(
        _node_pre_body,
        grid=(N // blk,),
        in_specs=[
            pl.BlockSpec((blk, D), lambda i: (i, 0)),
            pl.BlockSpec((blk, D), lambda i: (i, 0)),
            pl.BlockSpec((7, D, D), lambda i: (0, 0, 0)),
            pl.BlockSpec((7, 1, D), lambda i: (0, 0, 0)),
        ],
        out_specs=pl.BlockSpec((6, blk, D), lambda i: (0, i, 0)),
        out_shape=jax.ShapeDtypeStruct((6, N, D), jnp.float32),
    )(h, p, W, B)


def _edge_mm_body(e_ref, w_ref, b_ref, out_ref):
    out_ref[...] = jax.lax.dot_general(
        e_ref[...], w_ref[...], (((1,), (0,)), ((), ())),
        preferred_element_type=jnp.float32) + b_ref[...]


def _edge_matmul(e, W, b):
    blk = 4000
    return pl.pallas_call(
        _edge_mm_body,
        grid=(E // blk,),
        in_specs=[
            pl.BlockSpec((blk, D), lambda i: (i, 0)),
            pl.BlockSpec((D, D), lambda i: (0, 0)),
            pl.BlockSpec((1, D), lambda i: (0, 0)),
        ],
        out_specs=pl.BlockSpec((blk, D), lambda i: (i, 0)),
        out_shape=jax.ShapeDtypeStruct((E, D), jnp.float32),
    )(e, W, b.reshape(1, D))


def _clamp_body(d_ref, o0_ref, o1_ref):
    d = d_ref[...]
    scratch = NH + jnp.bitwise_and(d, 63)
    o0_ref[...] = jnp.where(d < NH, d, scratch)
    o1_ref[...] = jnp.where(d >= NH, d - NH, scratch)


def _clamp_dst(dst):
    d2 = dst.reshape(E // 128, 128)
    blk = E // 128
    assert (E // 128) % blk == 0
    o0, o1 = pl.pallas_call(
        _clamp_body,
        grid=(E // 128 // blk,),
        in_specs=[pl.BlockSpec((blk, 128), lambda i: (i, 0))],
        out_specs=[pl.BlockSpec((blk, 128), lambda i: (i, 0))] * 2,
        out_shape=[jax.ShapeDtypeStruct((E // 128, 128), jnp.int32)] * 2,
    )(d2)
    return o0.reshape(E), o1.reshape(E)


def _estats_body(x_ref, out_ref):
    i = pl.program_id(0)

    @pl.when(i == 0)
    def _():
        out_ref[...] = jnp.zeros_like(out_ref)

    x = x_ref[...]
    s = jnp.sum(x, axis=0, keepdims=True)
    s2 = jnp.sum(x * x, axis=0, keepdims=True)
    out_ref[...] += jnp.concatenate([s, s2], axis=0)


def _edge_stats(hat):
    blk = 8000
    return pl.pallas_call(
        _estats_body,
        grid=(E // blk,),
        in_specs=[pl.BlockSpec((blk, D), lambda i: (i, 0))],
        out_specs=pl.BlockSpec((2, D), lambda i: (0, 0)),
        out_shape=jax.ShapeDtypeStruct((2, D), jnp.float32),
    )(hat)


def _eapply_body(x_ref, st_ref, g_ref, b_ref, out_ref):
    m = st_ref[0:1] * (1.0 / E)
    var = st_ref[1:2] * (1.0 / E) - m * m
    inv = jax.lax.rsqrt(var + 1e-5)
    out_ref[...] = jnp.maximum((x_ref[...] - m) * inv * g_ref[...] + b_ref[...], 0.0)


def _edge_apply(hat, stats, g, b):
    blk = 8000
    return pl.pallas_call(
        _eapply_body,
        grid=(E // blk,),
        in_specs=[
            pl.BlockSpec((blk, D), lambda i: (i, 0)),
            pl.BlockSpec((2, D), lambda i: (0, 0)),
            pl.BlockSpec((1, D), lambda i: (0, 0)),
            pl.BlockSpec((1, D), lambda i: (0, 0)),
        ],
        out_specs=pl.BlockSpec((blk, D), lambda i: (i, 0)),
        out_shape=jax.ShapeDtypeStruct((E, D), jnp.float32),
    )(hat, stats, g.reshape(1, D), b.reshape(1, D))


def _combine_body(p_ref, out_ref):
    j = pl.program_id(0)
    out_ref[...] = p_ref[0, 0] + p_ref[1, 0]


def _combine(parts):
    return pl.pallas_call(
        _combine_body,
        grid=(2,),
        in_specs=[pl.BlockSpec((NC, 1, NH, D), lambda j: (0, j, 0, 0))],
        out_specs=pl.BlockSpec((NH, D), lambda j: (j, 0)),
        out_shape=jax.ShapeDtypeStruct((N, D), jnp.float32),
    )(parts)


def _nodefin_body(a1_ref, c1_ref, s_ref, sv_ref, sp_ref, g_ref, b_ref,
                  h_ref, p_ref):
    denom = s_ref[...] + 1e-6
    h_pre = a1_ref[...] + sv_ref[...] / denom
    p_pre = c1_ref[...] + sp_ref[...] / denom
    m = jnp.mean(h_pre, axis=0, keepdims=True)
    var = jnp.mean(h_pre * h_pre, axis=0, keepdims=True) - m * m
    inv = jax.lax.rsqrt(var + 1e-5)
    h_ref[...] = jnp.maximum((h_pre - m) * inv * g_ref[...] + b_ref[...], 0.0)
    p_ref[...] = jnp.tanh(p_pre)


def _node_finalize(A1_h, C1p, S, SV, SP, g, b):
    return pl.pallas_call(
        _nodefin_body,
        in_specs=[pl.BlockSpec((N, D), lambda: (0, 0))] * 5
        + [pl.BlockSpec((1, D), lambda: (0, 0))] * 2,
        out_specs=[pl.BlockSpec((N, D), lambda: (0, 0))] * 2,
        out_shape=[jax.ShapeDtypeStruct((N, D), jnp.float32)] * 2,
    )(A1_h, C1p, S, SV, SP, g.reshape(1, D), b.reshape(1, D))


_RCH = 80
_RPT = PAD // NS

CH = 80
EPW = E // NW
NCH = EPW // CH


_ZROWS = 16


def _zero_shared(acc, zbuf, tid):
    @pl.loop(0, _ZROWS)
    def _(r):
        for v in range(NV):
            zbuf[r, pl.ds(v * 16, 16)] = jnp.zeros((16,), jnp.float32)

    @pl.loop(0, _RPT // _ZROWS)
    def _(j):
        pltpu.sync_copy(zbuf, acc.at[pl.ds(tid * _RPT + j * _ZROWS, _ZROWS)])


def _writeout_shared(acc, out_slot, tid):
    @pl.loop(0, _RPT // _RCH)
    def _(j):
        row0 = tid * _RPT + j * _RCH
        pltpu.sync_copy(acc.at[pl.ds(row0, _RCH)], out_slot.at[pl.ds(row0, _RCH)])


_NBUF = 2


def _sc_ctx(src_hbm, d0_hbm, d1_hbm, sig_hbm, scratch):
    (sb0, sb1, db0, db1, cb0, cb1,
     xb0, xb1, g10, g11, g20, g21,
     acc, zbuf, si0, si1, so0, so1) = scratch
    ctx = {}
    c = jax.lax.axis_index("c")
    s = jax.lax.axis_index("s")
    wid = c * NS + s
    srcb = (sb0, sb1)
    dstb = (db0, db1)
    scatb = (cb0, cb1)
    xb = (xb0, xb1)
    g1 = (g10, g11)
    g2 = (g20, g21)
    semi = (si0, si1)
    semo = (so0, so1)
    dhbm = (d0_hbm, d1_hbm)

    def writeout1(out_slot):
        @pl.when(c == 0)
        def _():
            _writeout_shared(acc, out_slot.at[0], s)

        @pl.when(c == 1)
        def _():
            _writeout_shared(acc, out_slot.at[1], s)

    def run_phase(issue_in, wait_in, compute, issue_out, drain_out):
        issue_in(0, 0)

        @pl.loop(0, NCH + (-NCH) % _NBUF, step=_NBUF)
        def _(k0):
            for i in range(_NBUF):
                b = i
                k = k0 + i

                @pl.when(k < NCH)
                def _():
                    bn = 1 - b

                    @pl.when(k >= 1)
                    def _():
                        drain_out(bn, k - 1)

                    @pl.when(k + 1 < NCH)
                    def _():
                        issue_in(bn, k + 1)

                    wait_in(b, k)
                    compute(b)
                    issue_out(b, k)

        drain_out((NCH - 1) % _NBUF, NCH - 1)

    def hat_phase(b3e_hbm, b1_hbm, b2_hbm, hat_hbm, out_slot, half):
        _zero_shared(acc, zbuf, s)
        plsc.subcore_barrier()

        def issue_in(b, k):
            base = wid * EPW + k * CH
            pltpu.sync_copy(src_hbm.at[pl.ds(base, CH)], srcb[b])
            pltpu.sync_copy(dhbm[half].at[pl.ds(base, CH)], scatb[b])
            pltpu.async_copy(b3e_hbm.at[pl.ds(base, CH)], xb[b], semi[b])
            pltpu.async_copy(b1_hbm.at[srcb[b]], g1[b], semi[b])
            pltpu.async_copy(b2_hbm.at[dstb[b]], g2[b], semi[b])

        def wait_in(b, k):
            base = wid * EPW + k * CH
            pltpu.make_async_copy(b3e_hbm.at[pl.ds(base, CH)], xb[b], semi[b]).wait()
            pltpu.make_async_copy(b1_hbm.at[srcb[b]], g1[b], semi[b]).wait()
            pltpu.make_async_copy(b2_hbm.at[dstb[b]], g2[b], semi[b]).wait()

        def compute(b):
            xbb = xb[b]
            g1b = g1[b]
            g2b = g2[b]

            @pl.loop(0, CH, step=2)
            def _(r0):
                for dr in range(2):
                    r = r0 + dr
                    for v in range(NV):
                        sl = pl.ds(v * 16, 16)
                        x = xbb[r, sl] + g1b[r, sl] + g2b[r, sl]
                        xbb[r, sl] = x
                        g1b[r, sl] = 1.0 / (1.0 + jnp.exp(-x))

        def issue_out(b, k):
            base = wid * EPW + k * CH
            pltpu.async_copy(xb[b], hat_hbm.at[pl.ds(base, CH)], semo[b])
            pltpu.async_copy(g1[b], sig_hbm.at[pl.ds(base, CH)], semo[b])
            pltpu.sync_copy(g1[b], acc.at[scatb[b]], add=True)

        def drain_out(b, k):
            base = wid * EPW + k * CH
            pltpu.make_async_copy(xb[b], hat_hbm.at[pl.ds(base, CH)], semo[b]).wait()
            pltpu.make_async_copy(g1[b], sig_hbm.at[pl.ds(base, CH)], semo[b]).wait()

        def issue_in2(b, k):
            base = wid * EPW + k * CH
            pltpu.sync_copy(dst_ref[0].at[pl.ds(base, CH)], dstb[b])
            issue_in(b, k)

        run_phase(issue_in2, wait_in, compute, issue_out, drain_out)
        plsc.subcore_barrier()
        writeout1(out_slot)
        plsc.subcore_barrier()

    def sig_phase(out_slot, half):
        _zero_shared(acc, zbuf, s)
        plsc.subcore_barrier()

        def issue_in(b, k):
            base = wid * EPW + k * CH
            pltpu.sync_copy(dhbm[half].at[pl.ds(base, CH)], scatb[b])
            pltpu.async_copy(sig_hbm.at[pl.ds(base, CH)], g1[b], semi[b])

        def wait_in(b, k):
            base = wid * EPW + k * CH
            pltpu.make_async_copy(sig_hbm.at[pl.ds(base, CH)], g1[b], semi[b]).wait()

        def compute(b):
            pass

        def issue_out(b, k):
            pltpu.sync_copy(g1[b], acc.at[scatb[b]], add=True)

        def drain_out(b, k):
            pass

        run_phase(issue_in, wait_in, compute, issue_out, drain_out)
        plsc.subcore_barrier()
        writeout1(out_slot)
        plsc.subcore_barrier()

    def gate_phase(tab_hbm, out_slot, half):
        _zero_shared(acc, zbuf, s)
        plsc.subcore_barrier()

        def issue_in(b, k):
            base = wid * EPW + k * CH
            pltpu.sync_copy(src_hbm.at[pl.ds(base, CH)], srcb[b])
            pltpu.sync_copy(dhbm[half].at[pl.ds(base, CH)], scatb[b])
            pltpu.async_copy(sig_hbm.at[pl.ds(base, CH)], xb[b], semi[b])
            pltpu.async_copy(tab_hbm.at[srcb[b]], g1[b], semi[b])

        def wait_in(b, k):
            base = wid * EPW + k * CH
            pltpu.make_async_copy(sig_hbm.at[pl.ds(base, CH)], xb[b], semi[b]).wait()
            pltpu.make_async_copy(tab_hbm.at[srcb[b]], g1[b], semi[b]).wait()

        def compute(b):
            xbb = xb[b]
            g1b = g1[b]

            @pl.loop(0, CH, step=2)
            def _(r0):
                for dr in range(2):
                    r = r0 + dr
                    for v in range(NV):
                        sl = pl.ds(v * 16, 16)
                        g1b[r, sl] = g1b[r, sl] * xbb[r, sl]

        def issue_out(b, k):
            pltpu.sync_copy(g1[b], acc.at[scatb[b]], add=True)

        def drain_out(b, k):
            pass

        run_phase(issue_in, wait_in, compute, issue_out, drain_out)
        plsc.subcore_barrier()
        writeout1(out_slot)
        plsc.subcore_barrier()

    dst_ref = [None]
    ctx["hat_phase"] = hat_phase
    ctx["sig_phase"] = sig_phase
    ctx["gate_phase"] = gate_phase
    ctx["dst_ref"] = dst_ref
    return ctx


_SCRATCH = ([pltpu.VMEM((CH,), jnp.int32)] * 6
            + [pltpu.VMEM((CH, D), jnp.float32)] * 6
            + [
                pltpu.VMEM_SHARED((PAD, D), jnp.float32),
                pltpu.VMEM((_ZROWS, D), jnp.float32),
            ]
            + [pltpu.SemaphoreType.DMA] * 4)


def _sc_hat_body(b3e_hbm, src_hbm, dst_hbm, d0_hbm, b1_hbm, b2_hbm,
                 hat_hbm, sig_hbm, sigp0_hbm, *scratch):
    ctx = _sc_ctx(src_hbm, d0_hbm, d0_hbm, sig_hbm, scratch)
    ctx["dst_ref"][0] = dst_hbm
    ctx["hat_phase"](b3e_hbm, b1_hbm, b2_hbm, hat_hbm, sigp0_hbm, 0)


def _sc_gates_body(sig_hbm, src_hbm, d0_hbm, d1_hbm, v_hbm, cp_hbm,
                   sigp1_hbm, svp_hbm, spp_hbm, *scratch):
    ctx = _sc_ctx(src_hbm, d0_hbm, d1_hbm, sig_hbm, scratch)
    gate = ctx["gate_phase"]
    gate(v_hbm, svp_hbm.at[0], 0)
    gate(cp_hbm, spp_hbm.at[0], 0)
    ctx["sig_phase"](sigp1_hbm, 1)
    gate(v_hbm, svp_hbm.at[1], 1)
    gate(cp_hbm, spp_hbm.at[1], 1)


def _sc_hat(b3e, src, dst, d0, b1n, b2n):
    f = pl.kernel(
        _sc_hat_body,
        out_type=[jax.ShapeDtypeStruct((E, D), jnp.float32),
                  jax.ShapeDtypeStruct((E, D), jnp.float32),
                  jax.ShapeDtypeStruct((NC, PAD, D), jnp.float32)],
        mesh=_MESH,
        scratch_types=list(_SCRATCH),
    )
    return f(b3e, src, dst, d0, b1n, b2n)


def _sc_gates(sig, src, d0, d1, vtab, cptab):
    f = pl.kernel(
        _sc_gates_body,
        out_type=[jax.ShapeDtypeStruct((NC, PAD, D), jnp.float32),
                  jax.ShapeDtypeStruct((2, NC, PAD, D), jnp.float32),
                  jax.ShapeDtypeStruct((2, NC, PAD, D), jnp.float32)],
        mesh=_MESH,
        scratch_types=list(_SCRATCH),
    )
    return f(sig, src, d0, d1, vtab, cptab)


@jax.jit
def kernel(h, e, p, edge_index, A1_W, A1_b, A2_W, A2_b, B1_W, B1_b, B2_W, B2_b,
           B3_W, B3_b, C1_W, C1_b, C2_W, C2_b, bn_h_g, bn_h_b, bn_e_g, bn_e_b):
    src = edge_index[0]
    dst = edge_index[1]
    d0, d1 = _clamp_dst(dst)

    W = jnp.stack([A1_W, B1_W, B2_W, C1_W, A2_W[:D], A2_W[D:], C2_W])
    B = jnp.stack([A1_b, B1_b, B2_b, C1_b, A2_b, jnp.zeros_like(A2_b), C2_b])[:, None, :]

    nodes = _node_precompute(h, p, W, B)
    B3e = _edge_matmul(e, B3_W, B3_b)

    hat, sig, sigp0 = _sc_hat(B3e, src, dst, d0, nodes[1], nodes[2])
    sigp1, svp, spp = _sc_gates(sig, src, d0, d1, nodes[4], nodes[5])

    stats = _edge_stats(hat)
    e_new = _edge_apply(hat, stats, bn_e_g, bn_e_b)
    S = _combine(jnp.stack([sigp0, sigp1], axis=1))
    SV = _combine(jnp.transpose(svp, (1, 0, 2, 3)))
    SP = _combine(jnp.transpose(spp, (1, 0, 2, 3)))
    h_new, p_new = _node_finalize(nodes[0], nodes[3], S, SV, SP, bn_h_g, bn_h_b)
    return (h_new, e_new, p_new)
